# Initial kernel scaffold; baseline (speedup 1.0000x reference)
import jax
import jax.numpy as jnp
from jax.experimental import pallas as pl


def kernel(users, pos_items, neg_items, user_emb, item_emb):
    raise NotImplementedError("write your pallas kernel here")


# speedup vs baseline: 1.1072x; 1.1072x over previous
import functools

import jax
import jax.numpy as jnp
from jax import lax
from jax.experimental import pallas as pl
from jax.experimental.pallas import tpu as pltpu
from jax.experimental.pallas import tpu_sc as plsc

N_USERS = 100000
N_ITEMS = 100000
DIM = 128
B = 16384
DECAY = 1e-4

NC = 2
NS = 16
NW = NC * NS
BPW = B // NW
CHUNK = 128
NCHUNK = BPW // CHUNK
LANES = 16
VPR = DIM // LANES


def _sc_body(uemb, iemb, uidx, pidx, nidx,
             pos_out, neg_out, sq_out,
             idx_u, idx_p, idx_n, urows, prows, nrows, pbuf, nbuf, sqbuf, sem):
    wid = lax.axis_index("s") * NC + lax.axis_index("c")
    base = wid * BPW

    pltpu.sync_copy(uidx.at[wid], idx_u)
    pltpu.sync_copy(pidx.at[wid], idx_p)
    pltpu.sync_copy(nidx.at[wid], idx_n)

    zero = jnp.zeros((LANES,), jnp.float32)
    su = zero
    sp = zero
    sn = zero

    for c in range(NCHUNK):
        cp_u = pltpu.async_copy(uemb.at[idx_u.at[c]], urows, sem)
        cp_p = pltpu.async_copy(iemb.at[idx_p.at[c]], prows, sem)
        cp_n = pltpu.async_copy(iemb.at[idx_n.at[c]], nrows, sem)
        cp_u.wait()
        cp_p.wait()
        cp_n.wait()

        def row_body(r, carry):
            su, sp, sn = carry
            accp = zero
            accn = zero
            for j in range(VPR):
                uv = urows[r, pl.ds(j * LANES, LANES)]
                pv = prows[r, pl.ds(j * LANES, LANES)]
                nv = nrows[r, pl.ds(j * LANES, LANES)]
                accp = accp + uv * pv
                accn = accn + uv * nv
                su = su + uv * uv
                sp = sp + pv * pv
                sn = sn + nv * nv
            pbuf[r, :] = accp
            nbuf[r, :] = accn
            return su, sp, sn

        su, sp, sn = lax.fori_loop(0, CHUNK, row_body, (su, sp, sn))
        pltpu.sync_copy(pbuf, pos_out.at[pl.ds(base + c * CHUNK, CHUNK)])
        pltpu.sync_copy(nbuf, neg_out.at[pl.ds(base + c * CHUNK, CHUNK)])

    sqbuf[0, :] = su
    sqbuf[1, :] = sp
    sqbuf[2, :] = sn
    pltpu.sync_copy(sqbuf, sq_out.at[wid])


def _tc_body(pos_ref, neg_ref, sq_ref, out_ref):
    pos = pos_ref[...]
    neg = neg_ref[...]
    kk = lax.broadcasted_iota(jnp.int32, (DIM, DIM // LANES), 0) // LANES
    jj = lax.broadcasted_iota(jnp.int32, (DIM, DIM // LANES), 1)
    sel = (kk == jj).astype(jnp.float32)
    ps = jnp.dot(pos, sel, preferred_element_type=jnp.float32)
    ns = jnp.dot(neg, sel, preferred_element_type=jnp.float32)
    d = ps - ns
    bpr = jnp.sum(-jnp.log(1e-5 + jax.nn.sigmoid(d))) / B
    sq = sq_ref[...]
    s_u = jnp.sum(sq[:, 0:16])
    s_p = jnp.sum(sq[:, 16:32])
    s_n = jnp.sum(sq[:, 32:48])
    emb = (jnp.sqrt(s_u) + jnp.sqrt(s_p) + jnp.sqrt(s_n)) / B * DECAY
    out_ref[...] = jnp.reshape(bpr + emb / B, (1, 1))


@jax.jit
def kernel(users, pos_items, neg_items, user_emb, item_emb):
    uidx = users.astype(jnp.int32).reshape(NW, NCHUNK, CHUNK)
    pidx = pos_items.astype(jnp.int32).reshape(NW, NCHUNK, CHUNK)
    nidx = neg_items[:, 0].astype(jnp.int32).reshape(NW, NCHUNK, CHUNK)

    sc = pl.kernel(
        _sc_body,
        mesh=plsc.VectorSubcoreMesh(core_axis_name="c", subcore_axis_name="s"),
        out_type=(
            jax.ShapeDtypeStruct((B, LANES), jnp.float32),
            jax.ShapeDtypeStruct((B, LANES), jnp.float32),
            jax.ShapeDtypeStruct((NW, 3, LANES), jnp.float32),
        ),
        scratch_types=[
            pltpu.VMEM((NCHUNK, CHUNK), jnp.int32),
            pltpu.VMEM((NCHUNK, CHUNK), jnp.int32),
            pltpu.VMEM((NCHUNK, CHUNK), jnp.int32),
            pltpu.VMEM((CHUNK, DIM), jnp.float32),
            pltpu.VMEM((CHUNK, DIM), jnp.float32),
            pltpu.VMEM((CHUNK, DIM), jnp.float32),
            pltpu.VMEM((CHUNK, LANES), jnp.float32),
            pltpu.VMEM((CHUNK, LANES), jnp.float32),
            pltpu.VMEM((3, LANES), jnp.float32),
            pltpu.SemaphoreType.DMA,
        ],
    )
    pos_part, neg_part, sq = sc(user_emb, item_emb, uidx, pidx, nidx)

    pos2 = pos_part.reshape(B * LANES // DIM, DIM)
    neg2 = neg_part.reshape(B * LANES // DIM, DIM)
    sq2 = sq.reshape(NW, 3 * LANES)

    res = pl.pallas_call(
        _tc_body,
        out_shape=jax.ShapeDtypeStruct((1, 1), jnp.float32),
    )(pos2, neg2, sq2)

    s = res[0, 0]
    return (s, s, s)


# speedup vs baseline: 1.7524x; 1.5828x over previous
import functools

import jax
import jax.numpy as jnp
from jax import lax
from jax.experimental import pallas as pl
from jax.experimental.pallas import tpu as pltpu
from jax.experimental.pallas import tpu_sc as plsc

N_USERS = 100000
N_ITEMS = 100000
DIM = 128
B = 16384
DECAY = 1e-4

NC = 2
NS = 16
NW = NC * NS
BPW = B // NW
CHUNK = 128
NCHUNK = BPW // CHUNK
LANES = 16
VPR = DIM // LANES
PROWS = CHUNK * LANES // DIM


def _sc_body(uemb, iemb, uidx, pidx, nidx,
             pos_out, neg_out, sq_out,
             idx_u, idx_p, idx_n, urows, prows, nrows, pbuf, nbuf, sqbuf, sem):
    wid = lax.axis_index("s") * NC + lax.axis_index("c")
    base = wid * BPW

    pltpu.sync_copy(uidx.at[pl.ds(base, BPW)], idx_u)
    pltpu.sync_copy(pidx.at[pl.ds(base, BPW)], idx_p)
    pltpu.sync_copy(nidx.at[pl.ds(base, BPW)], idx_n)

    zero = jnp.zeros((LANES,), jnp.float32)

    def gather(c, buf):
        s = pl.ds(c * CHUNK, CHUNK)
        return (pltpu.async_copy(uemb.at[idx_u.at[s]], urows.at[buf], sem),
                pltpu.async_copy(iemb.at[idx_p.at[s]], prows.at[buf], sem),
                pltpu.async_copy(iemb.at[idx_n.at[s]], nrows.at[buf], sem))

    cps = gather(0, 0)
    su = zero
    sp = zero
    sn = zero

    for c in range(NCHUNK):
        for cp in cps:
            cp.wait()
        if c + 1 < NCHUNK:
            nxt = gather(c + 1, (c + 1) % 2)
        buf = c % 2

        def row_body(r, carry):
            su, sp, sn = carry
            accp = zero
            accn = zero
            for j in range(VPR):
                uv = urows[buf, r, pl.ds(j * LANES, LANES)]
                pv = prows[buf, r, pl.ds(j * LANES, LANES)]
                nv = nrows[buf, r, pl.ds(j * LANES, LANES)]
                accp = accp + uv * pv
                accn = accn + uv * nv
                su = su + uv * uv
                sp = sp + pv * pv
                sn = sn + nv * nv
            pbuf[r // VPR, pl.ds((r % VPR) * LANES, LANES)] = accp
            nbuf[r // VPR, pl.ds((r % VPR) * LANES, LANES)] = accn
            return su, sp, sn

        su, sp, sn = lax.fori_loop(0, CHUNK, row_body, (su, sp, sn))
        out_row = wid * (BPW * LANES // DIM) + c * PROWS
        pltpu.sync_copy(pbuf, pos_out.at[pl.ds(out_row, PROWS)])
        pltpu.sync_copy(nbuf, neg_out.at[pl.ds(out_row, PROWS)])
        if c + 1 < NCHUNK:
            cps = nxt

    sqbuf[pl.ds(0, LANES)] = su
    sqbuf[pl.ds(LANES, LANES)] = sp
    sqbuf[pl.ds(2 * LANES, LANES)] = sn
    pltpu.sync_copy(sqbuf, sq_out.at[wid])


def _tc_body(pos_ref, neg_ref, sq_ref, out_ref):
    pos = pos_ref[...]
    neg = neg_ref[...]
    kk = lax.broadcasted_iota(jnp.int32, (DIM, DIM // LANES), 0) // LANES
    jj = lax.broadcasted_iota(jnp.int32, (DIM, DIM // LANES), 1)
    sel = (kk == jj).astype(jnp.float32)
    ps = jnp.dot(pos, sel, preferred_element_type=jnp.float32)
    ns = jnp.dot(neg, sel, preferred_element_type=jnp.float32)
    d = ps - ns
    bpr = jnp.sum(-jnp.log(1e-5 + jax.nn.sigmoid(d))) / B
    sq = sq_ref[...]
    s_u = jnp.sum(sq[:, 0:16])
    s_p = jnp.sum(sq[:, 16:32])
    s_n = jnp.sum(sq[:, 32:48])
    emb = (jnp.sqrt(s_u) + jnp.sqrt(s_p) + jnp.sqrt(s_n)) / B * DECAY
    out_ref[...] = jnp.reshape(bpr + emb / B, (1, 1))


@jax.jit
def kernel(users, pos_items, neg_items, user_emb, item_emb):
    uidx = users.astype(jnp.int32)
    pidx = pos_items.astype(jnp.int32)
    nidx = neg_items[:, 0].astype(jnp.int32)

    sc = pl.kernel(
        _sc_body,
        mesh=plsc.VectorSubcoreMesh(core_axis_name="c", subcore_axis_name="s"),
        out_type=(
            jax.ShapeDtypeStruct((B * LANES // DIM, DIM), jnp.float32),
            jax.ShapeDtypeStruct((B * LANES // DIM, DIM), jnp.float32),
            jax.ShapeDtypeStruct((NW, 3 * LANES), jnp.float32),
        ),
        scratch_types=[
            pltpu.VMEM((BPW,), jnp.int32),
            pltpu.VMEM((BPW,), jnp.int32),
            pltpu.VMEM((BPW,), jnp.int32),
            pltpu.VMEM((2, CHUNK, DIM), jnp.float32),
            pltpu.VMEM((2, CHUNK, DIM), jnp.float32),
            pltpu.VMEM((2, CHUNK, DIM), jnp.float32),
            pltpu.VMEM((PROWS, DIM), jnp.float32),
            pltpu.VMEM((PROWS, DIM), jnp.float32),
            pltpu.VMEM((3 * LANES,), jnp.float32),
            pltpu.SemaphoreType.DMA,
        ],
    )
    pos_part, neg_part, sq = sc(user_emb, item_emb, uidx, pidx, nidx)

    res = pl.pallas_call(
        _tc_body,
        out_shape=jax.ShapeDtypeStruct((1, 1), jnp.float32),
    )(pos_part, neg_part, sq)

    s = res[0, 0]
    return (s, s, s)
